# baseline (speedup 1.0000x reference)
import functools

import jax
import jax.numpy as jnp
from jax.experimental import pallas as pl
from jax.experimental.pallas import tpu as pltpu

_BN_EPS = 1e-5
_SN_EPS = 1e-12


def _fold_weights(w, u_row):
    Cin, Cout = w.shape[2], w.shape[3]
    wf = w.reshape(9 * Cin, Cout)
    v = jnp.sum(wf * u_row, axis=1, keepdims=True)
    v = v / (jnp.sqrt(jnp.sum(v * v)) + _SN_EPS)
    wv = jnp.sum(wf * v, axis=0, keepdims=True)
    u_new = wv / (jnp.sqrt(jnp.sum(wv * wv)) + _SN_EPS)
    sigma = jnp.sum(u_new * wv)
    w = w / sigma
    d0, d1, d2 = w[:, 0], w[:, 1], w[:, 2]
    z = jnp.zeros_like(d0)
    e = (jnp.concatenate([d0, z], axis=2),
         jnp.concatenate([d1 + d2, d0 + d1], axis=2),
         jnp.concatenate([z, d2], axis=2))
    pieces = []
    for pi in range(2):
        for bb in range(3):
            if pi == 0:
                t0, t1 = e[bb][0], e[bb][1] + e[bb][2]
            else:
                t0, t1 = e[bb][0] + e[bb][1], e[bb][2]
            pieces.append(jnp.concatenate([t0, t1], axis=0))
    return jnp.stack(pieces, axis=0).astype(jnp.bfloat16)


def _conv_accs(x_ref, w_ref, *, NB, H, W, Cin):
    HW = H * W
    S = HW + 2 * W
    M = NB * S
    z = jnp.zeros((W, Cin), jnp.float32)
    pieces = []
    for b in range(NB):
        pieces += [z, x_ref[b].reshape(HW, Cin), z]
    xe = jnp.concatenate(pieces, axis=0)
    xc = jnp.concatenate([xe, jnp.zeros((W, Cin), jnp.float32)], axis=0)
    xc = jnp.concatenate([xc[0:M], xc[W:M + W]], axis=1)
    col = jax.lax.broadcasted_iota(jnp.int32, (M, 2 * Cin), 0)
    col = (col & (W - 1)) if (W & (W - 1)) == 0 else (col % W)
    p1 = xc.astype(jnp.bfloat16)
    p0 = jnp.where(col != 0, pltpu.roll(xc, 1, axis=0), 0.0
                   ).astype(jnp.bfloat16)
    p2 = jnp.where(col != W - 1, pltpu.roll(xc, M - 1, axis=0), 0.0
                   ).astype(jnp.bfloat16)
    span = (NB - 1) * S + HW
    accs = [[None, None] for _ in range(NB)]
    for pi in range(2):
        acc = None
        for bb, p in ((0, p0), (1, p1), (2, p2)):
            d = jnp.dot(p[pi * W:pi * W + span], w_ref[pi * 3 + bb],
                        preferred_element_type=jnp.float32)
            acc = d if acc is None else acc + d
        for b in range(NB):
            accs[b][pi] = acc[b * S:b * S + HW]
    return accs


def _stats_kernel(x_ref, cw_ref, u_ref, st_ref, w2_ref,
                  *, NB, H, W, Cin, C2):
    n = pl.program_id(0)

    @pl.when(n == 0)
    def _():
        w2_ref[...] = _fold_weights(cw_ref[...], u_ref[...])
        st_ref[...] = jnp.zeros_like(st_ref)

    accs = _conv_accs(x_ref, w2_ref, NB=NB, H=H, W=W, Cin=Cin)
    ssum = None
    ssq = None
    for b in range(NB):
        for pi in range(2):
            a = accs[b][pi]
            s1 = jnp.sum(a, axis=0, keepdims=True)
            s2 = jnp.sum(a * a, axis=0, keepdims=True)
            ssum = s1 if ssum is None else ssum + s1
            ssq = s2 if ssq is None else ssq + s2
    st_ref[...] += jnp.concatenate([ssum, ssq], axis=0)


def _apply_kernel(x_ref, w_ref, st_ref, g_ref, bt_ref, o_ref,
                  *, NB, H, W, Cin, C2, cnt):
    Cout = C2 // 2
    st = st_ref[...]
    tot = st[:, 0:Cout] + st[:, Cout:C2]
    m0 = tot[0:1] * (1.0 / cnt)
    var = jnp.maximum(tot[1:2] * (1.0 / cnt) - m0 * m0, 0.0)
    invstd = jax.lax.rsqrt(var + _BN_EPS)
    scale = g_ref[...] * invstd
    shift = bt_ref[...] - m0 * scale
    sc = jnp.concatenate([scale, scale], axis=1)
    sh = jnp.concatenate([shift, shift], axis=1)

    accs = _conv_accs(x_ref, w_ref, NB=NB, H=H, W=W, Cin=Cin)
    for b in range(NB):
        for pi in range(2):
            o = jnp.maximum(accs[b][pi] * sc + sh, 0.0)
            for pj in range(2):
                opj = o[:, pj * Cout:(pj + 1) * Cout].reshape(H, W, Cout)
                o_ref[b, pl.ds(pi, H, 2), pl.ds(pj, W, 2)] = opj


def kernel(x, conv_w, conv_b, sn_u, bn_gamma, bn_beta):
    N, H, W, Cin = x.shape
    Cout = conv_w.shape[-1]
    H2, W2 = 2 * H, 2 * W
    C2 = 2 * Cout
    NB1 = 4 if N % 4 == 0 else (2 if N % 2 == 0 else 1)
    NB2 = 2 if N % 2 == 0 else 1

    u_row = sn_u.reshape(1, Cout)
    g_row = bn_gamma.reshape(1, Cout)
    bt_row = bn_beta.reshape(1, Cout)

    vmem_limit = 100 * 1024 * 1024

    stats, w2 = pl.pallas_call(
        functools.partial(_stats_kernel, NB=NB1, H=H, W=W, Cin=Cin, C2=C2),
        out_shape=(
            jax.ShapeDtypeStruct((2, C2), jnp.float32),
            jax.ShapeDtypeStruct((6, 2 * Cin, C2), jnp.bfloat16),
        ),
        grid=(N // NB1,),
        in_specs=[
            pl.BlockSpec((NB1, H, W, Cin), lambda n: (n, 0, 0, 0)),
            pl.BlockSpec((3, 3, Cin, Cout), lambda n: (0, 0, 0, 0)),
            pl.BlockSpec((1, Cout), lambda n: (0, 0)),
        ],
        out_specs=(
            pl.BlockSpec((2, C2), lambda n: (0, 0)),
            pl.BlockSpec((6, 2 * Cin, C2), lambda n: (0, 0, 0)),
        ),
        compiler_params=pltpu.CompilerParams(
            dimension_semantics=("arbitrary",),
            vmem_limit_bytes=vmem_limit),
    )(x, conv_w, u_row)

    out = pl.pallas_call(
        functools.partial(_apply_kernel, NB=NB2, H=H, W=W, Cin=Cin, C2=C2,
                          cnt=float(N * H2 * W2)),
        out_shape=jax.ShapeDtypeStruct((N, H2, W2, Cout), jnp.float32),
        grid=(N // NB2,),
        in_specs=[
            pl.BlockSpec((NB2, H, W, Cin), lambda n: (n, 0, 0, 0)),
            pl.BlockSpec((6, 2 * Cin, C2), lambda n: (0, 0, 0)),
            pl.BlockSpec((2, C2), lambda n: (0, 0)),
            pl.BlockSpec((1, Cout), lambda n: (0, 0)),
            pl.BlockSpec((1, Cout), lambda n: (0, 0)),
        ],
        out_specs=pl.BlockSpec((NB2, H2, W2, Cout), lambda n: (n, 0, 0, 0)),
        compiler_params=pltpu.CompilerParams(
            dimension_semantics=("arbitrary",),
            vmem_limit_bytes=vmem_limit),
    )(x, w2, stats, g_row, bt_row)

    return out
